# baseline (speedup 1.0000x reference)
import functools

import jax
import jax.numpy as jnp
from jax import lax
from jax.experimental import pallas as pl
from jax.experimental.pallas import tpu as pltpu
from jax.experimental.pallas import tpu_sc as plsc

N = 10000
E = 320000
H = 128
R = 32
G = 64

NC = 2
NS = 16
NW = NC * NS

NP = 10240
CH = 80
GP = 128

_MESH = dict(core_axis_name="c", subcore_axis_name="s")


def _wid():
    return lax.axis_index("s") * NC + lax.axis_index("c")


def _make_gather(B, depth):
    assert B % (NW * CH) == 0
    chunks = B // (NW * CH)
    per_w = chunks * CH
    assert chunks % depth == 0 and chunks // depth >= 2

    @functools.partial(
        pl.kernel,
        out_type=jax.ShapeDtypeStruct((B, H), jnp.float32),
        mesh=plsc.VectorSubcoreMesh(**_MESH),
        scratch_types=[
            pltpu.VMEM((per_w,), jnp.int32),
            *([pltpu.VMEM((CH, H), jnp.float32)] * depth),
            *([pltpu.SemaphoreType.DMA] * depth),
        ],
    )
    def gather_k(table, idx, out, idx_v, *bufs):
        rb = bufs[:depth]
        sems = bufs[depth:]
        base = _wid() * per_w
        pltpu.sync_copy(idx.at[pl.ds(base, per_w)], idx_v)

        def start(i, b):
            pltpu.async_copy(
                table.at[idx_v.at[pl.ds(i * CH, CH)]], rb[b], sems[b]
            )

        def wait(b):
            pltpu.make_async_copy(
                table.at[idx_v.at[pl.ds(0, CH)]], rb[b], sems[b]
            ).wait()

        for b in range(depth):
            start(b, b)

        @pl.loop(0, chunks // depth - 1)
        def _(j):
            k = j * depth
            for b in range(depth):
                wait(b)
                pltpu.sync_copy(rb[b], out.at[pl.ds(base + (k + b) * CH, CH)])
                start(k + b + depth, b)

        k_last = chunks - depth
        for b in range(depth):
            wait(b)
            pltpu.sync_copy(rb[b], out.at[pl.ds(base + (k_last + b) * CH, CH)])

    return gather_k


def _make_scatter(B, nseg, depth, ch):
    assert B % (NW * ch) == 0 and nseg % (NS * 8) == 0
    chunks = B // (NW * ch)
    per_w = chunks * ch
    assert chunks % depth == 0 and chunks // depth >= 2
    rpt = nseg // NS

    @functools.partial(
        pl.kernel,
        out_type=jax.ShapeDtypeStruct((NC, nseg, H), jnp.float32),
        mesh=plsc.VectorSubcoreMesh(**_MESH),
        scratch_types=[
            *([pltpu.VMEM((ch,), jnp.int32)] * depth),
            *([pltpu.VMEM((ch, H), jnp.float32)] * depth),
            pltpu.VMEM_SHARED((nseg, H), jnp.float32),
            *([pltpu.SemaphoreType.DMA] * depth),
        ],
    )
    def scatter_k(vals, idx, zeros_c, out, *rest):
        ib = rest[:depth]
        vb = rest[depth:2 * depth]
        acc = rest[2 * depth]
        sems = rest[2 * depth + 1:]
        cid = lax.axis_index("c")
        sid = lax.axis_index("s")
        rbase = sid * rpt
        pltpu.sync_copy(zeros_c.at[pl.ds(0, ch)], vb[0])
        nz = (rpt + ch - 1) // ch
        for j in range(nz):
            rows = min(ch, rpt - j * ch)
            pltpu.sync_copy(
                vb[0].at[pl.ds(0, rows)], acc.at[pl.ds(rbase + j * ch, rows)]
            )
        plsc.subcore_barrier()

        base = _wid() * per_w

        def start(i, b):
            pltpu.sync_copy(idx.at[pl.ds(base + i * ch, ch)], ib[b])
            pltpu.async_copy(vals.at[pl.ds(base + i * ch, ch)], vb[b], sems[b])

        def wait(b):
            pltpu.make_async_copy(
                vals.at[pl.ds(base, ch)], vb[b], sems[b]
            ).wait()

        for b in range(depth):
            start(b, b)

        @pl.loop(0, chunks // depth - 1)
        def _(j):
            k = j * depth
            for b in range(depth):
                wait(b)
                pltpu.sync_copy(vb[b], acc.at[ib[b]], add=True)
                start(k + b + depth, b)

        for b in range(depth):
            wait(b)
            pltpu.sync_copy(vb[b], acc.at[ib[b]], add=True)

        plsc.subcore_barrier()
        pltpu.sync_copy(
            acc.at[pl.ds(rbase, rpt)], out.at[cid, pl.ds(rbase, rpt)]
        )

    return scatter_k


def _make_pool():
    chunks = NP // (NW * CH)
    per_w = chunks * CH
    rpt = GP // NS

    @functools.partial(
        pl.kernel,
        out_type=(
            jax.ShapeDtypeStruct((NC, GP, H), jnp.float32),
            jax.ShapeDtypeStruct((NC, GP, H), jnp.float32),
        ),
        mesh=plsc.VectorSubcoreMesh(**_MESH),
        scratch_types=[
            pltpu.VMEM((CH,), jnp.int32),
            pltpu.VMEM((CH, H), jnp.float32),
            pltpu.VMEM((CH, H), jnp.float32),
            pltpu.VMEM_SHARED((GP, H), jnp.float32),
            pltpu.VMEM_SHARED((GP, H), jnp.float32),
            pltpu.SemaphoreType.DMA,
        ],
    )
    def pool_k(vals, idx, zeros_c, ones_c, out_s, out_n, idx_v, vals_v,
               ones_v, acc_s, acc_n, sem):
        cid = lax.axis_index("c")
        sid = lax.axis_index("s")
        rbase = sid * rpt
        pltpu.sync_copy(zeros_c.at[pl.ds(0, CH)], vals_v)
        pltpu.sync_copy(ones_c.at[pl.ds(0, CH)], ones_v)
        pltpu.sync_copy(vals_v.at[pl.ds(0, rpt)], acc_s.at[pl.ds(rbase, rpt)])
        pltpu.sync_copy(vals_v.at[pl.ds(0, rpt)], acc_n.at[pl.ds(rbase, rpt)])
        plsc.subcore_barrier()

        base = _wid() * per_w

        @pl.loop(0, chunks)
        def _(i):
            off = base + i * CH
            pltpu.sync_copy(idx.at[pl.ds(off, CH)], idx_v)
            pltpu.sync_copy(vals.at[pl.ds(off, CH)], vals_v)
            pltpu.sync_copy(vals_v, acc_s.at[idx_v], add=True)
            pltpu.sync_copy(ones_v, acc_n.at[idx_v], add=True)

        plsc.subcore_barrier()
        pltpu.sync_copy(acc_s.at[pl.ds(rbase, rpt)], out_s.at[cid, pl.ds(rbase, rpt)])
        pltpu.sync_copy(acc_n.at[pl.ds(rbase, rpt)], out_n.at[cid, pl.ds(rbase, rpt)])

    return pool_k


CE = 640


def _edge_tc(hdhs2, ea, Wc, bc):
    nb = E // CE

    def body(hh_ref, ea_ref, w_ref, b_ref, m_ref):
        w = w_ref[...]
        hd = hh_ref[0].astype(jnp.bfloat16)
        hs = hh_ref[1].astype(jnp.bfloat16)
        eab = ea_ref[...].astype(jnp.bfloat16)
        acc = jnp.dot(hd, w[0:H], preferred_element_type=jnp.float32)
        acc += jnp.dot(hs, w[H:2 * H], preferred_element_type=jnp.float32)
        acc += jnp.dot(eab, w[2 * H:], preferred_element_type=jnp.float32)
        acc += b_ref[...]
        f = acc[:, :H]
        s = acc[:, H:]
        sig = 0.5 + 0.5 * jnp.tanh(0.5 * f)
        sp = jnp.maximum(s, 0.0) + jnp.log1p(jnp.exp(-jnp.abs(s)))
        m_ref[...] = sig * sp

    return pl.pallas_call(
        body,
        grid=(nb,),
        in_specs=[
            pl.BlockSpec((2, CE, H), lambda i: (0, i, 0)),
            pl.BlockSpec((CE, R), lambda i: (i, 0)),
            pl.BlockSpec((2 * H + R, 2 * H), lambda i: (0, 0)),
            pl.BlockSpec((1, 2 * H), lambda i: (0, 0)),
        ],
        out_specs=pl.BlockSpec((CE, H), lambda i: (i, 0)),
        out_shape=jax.ShapeDtypeStruct((E, H), jnp.float32),
    )(hdhs2, ea, Wc, bc)


CB = 1024


def _combine_tc(h, parts):
    nb = NP // CB

    def body(h_ref, p_ref, o_ref):
        o_ref[...] = jnp.maximum(h_ref[...] + p_ref[0] + p_ref[1], 0.0)

    return pl.pallas_call(
        body,
        grid=(nb,),
        in_specs=[
            pl.BlockSpec((CB, H), lambda i: (i, 0)),
            pl.BlockSpec((2, CB, H), lambda i: (0, i, 0)),
        ],
        out_specs=pl.BlockSpec((CB, H), lambda i: (i, 0)),
        out_shape=jax.ShapeDtypeStruct((NP, H), jnp.float32),
    )(h, parts)


def _final_tc(sums, cnts, Wl, bl):
    def body(s_ref, c_ref, w_ref, b_ref, o_ref):
        ssum = s_ref[0] + s_ref[1]
        cnt = c_ref[0] + c_ref[1]
        pooled = ssum / jnp.maximum(cnt, 1.0)
        o_ref[...] = (
            jnp.dot(pooled, w_ref[...], preferred_element_type=jnp.float32)
            + b_ref[...]
        )

    return pl.pallas_call(
        body,
        in_specs=[
            pl.BlockSpec((NC, GP, H), lambda: (0, 0, 0)),
            pl.BlockSpec((NC, GP, H), lambda: (0, 0, 0)),
            pl.BlockSpec((H, H), lambda: (0, 0)),
            pl.BlockSpec((1, H), lambda: (0, 0)),
        ],
        out_specs=pl.BlockSpec((GP, H), lambda: (0, 0)),
        out_shape=jax.ShapeDtypeStruct((GP, H), jnp.float32),
    )(sums, cnts, Wl, bl.reshape(1, H))


_gather_emb = _make_gather(NP, depth=2)
_gather_edges = _make_gather(2 * E, depth=5)
_scatter_edges = _make_scatter(E, NP, depth=5, ch=40)
_pool = _make_pool()


def kernel(x, edge_index, edge_attr, batch, emb, Wf1, bf1, Ws1, bs1, Wf2, bf2,
           Ws2, bs2, Wf3, bf3, Ws3, bs3, Wl, bl):
    x = x.astype(jnp.int32)
    src = edge_index[0].astype(jnp.int32)
    dst = edge_index[1].astype(jnp.int32)
    batch = batch.astype(jnp.int32)

    xpad = jnp.pad(x, (0, NP - N))
    bpad = jnp.pad(batch, (0, NP - N), constant_values=G)
    eidx = jnp.concatenate([dst, src])
    zeros_c = jnp.zeros((CH, H), jnp.float32)
    ones_c = jnp.ones((CH, H), jnp.float32)

    h = _gather_emb(emb, xpad)

    layers = ((Wf1, bf1, Ws1, bs1), (Wf2, bf2, Ws2, bs2), (Wf3, bf3, Ws3, bs3))
    for Wf, bf, Ws, bs in layers:
        Wc = jnp.concatenate([Wf, Ws], axis=1).astype(jnp.bfloat16)
        bc = jnp.concatenate([bf, bs]).reshape(1, 2 * H)
        hdhs = _gather_edges(h, eidx).reshape(2, E, H)
        m = _edge_tc(hdhs, edge_attr, Wc, bc)
        parts = _scatter_edges(m, dst, zeros_c)
        h = _combine_tc(h, parts)

    sums, cnts = _pool(h, bpad, zeros_c, ones_c)
    out = _final_tc(sums, cnts, Wl, bl)
    return out[:G]


# speedup vs baseline: 1.3670x; 1.3670x over previous
import functools

import jax
import jax.numpy as jnp
from jax import lax
from jax.experimental import pallas as pl
from jax.experimental.pallas import tpu as pltpu
from jax.experimental.pallas import tpu_sc as plsc

N = 10000
E = 320000
H = 128
R = 32
G = 64

NC = 2
NS = 16
NW = NC * NS

NP = 10240
CH = 80
GP = 128

_MESH = dict(core_axis_name="c", subcore_axis_name="s")


def _wid():
    return lax.axis_index("s") * NC + lax.axis_index("c")


def _make_gather(B, depth):
    assert B % (NW * CH) == 0
    chunks = B // (NW * CH)
    per_w = chunks * CH
    assert chunks % depth == 0 and chunks // depth >= 2

    @functools.partial(
        pl.kernel,
        out_type=jax.ShapeDtypeStruct((B, H), jnp.float32),
        mesh=plsc.VectorSubcoreMesh(**_MESH),
        scratch_types=[
            pltpu.VMEM((per_w,), jnp.int32),
            *([pltpu.VMEM((CH, H), jnp.float32)] * depth),
            *([pltpu.SemaphoreType.DMA] * depth),
        ],
    )
    def gather_k(table, idx, out, idx_v, *bufs):
        rb = bufs[:depth]
        sems = bufs[depth:]
        base = _wid() * per_w
        pltpu.sync_copy(idx.at[pl.ds(base, per_w)], idx_v)

        def start(i, b):
            pltpu.async_copy(
                table.at[idx_v.at[pl.ds(i * CH, CH)]], rb[b], sems[b]
            )

        def wait(b):
            pltpu.make_async_copy(
                table.at[idx_v.at[pl.ds(0, CH)]], rb[b], sems[b]
            ).wait()

        for b in range(depth):
            start(b, b)

        @pl.loop(0, chunks // depth - 1)
        def _(j):
            k = j * depth
            for b in range(depth):
                wait(b)
                pltpu.sync_copy(rb[b], out.at[pl.ds(base + (k + b) * CH, CH)])
                start(k + b + depth, b)

        k_last = chunks - depth
        for b in range(depth):
            wait(b)
            pltpu.sync_copy(rb[b], out.at[pl.ds(base + (k_last + b) * CH, CH)])

    return gather_k


def _make_scatter(B, nseg, depth, ch):
    assert B % (NW * ch) == 0 and nseg % (NS * 8) == 0
    chunks = B // (NW * ch)
    per_w = chunks * ch
    main = (chunks // depth) * depth
    assert main // depth >= 2
    rpt = nseg // NS

    @functools.partial(
        pl.kernel,
        out_type=jax.ShapeDtypeStruct((NC, nseg, H), jnp.float32),
        mesh=plsc.VectorSubcoreMesh(**_MESH),
        scratch_types=[
            *([pltpu.VMEM((ch,), jnp.int32)] * depth),
            *([pltpu.VMEM((ch, H), jnp.float32)] * depth),
            pltpu.VMEM_SHARED((nseg, H), jnp.float32),
            *([pltpu.SemaphoreType.DMA] * depth),
        ],
    )
    def scatter_k(vals, idx, zeros_c, out, *rest):
        ib = rest[:depth]
        vb = rest[depth:2 * depth]
        acc = rest[2 * depth]
        sems = rest[2 * depth + 1:]
        cid = lax.axis_index("c")
        sid = lax.axis_index("s")
        rbase = sid * rpt
        pltpu.sync_copy(zeros_c.at[pl.ds(0, ch)], vb[0])
        nz = (rpt + ch - 1) // ch
        for j in range(nz):
            rows = min(ch, rpt - j * ch)
            pltpu.sync_copy(
                vb[0].at[pl.ds(0, rows)], acc.at[pl.ds(rbase + j * ch, rows)]
            )
        plsc.subcore_barrier()

        base = _wid() * per_w

        def start(i, b):
            pltpu.sync_copy(idx.at[pl.ds(base + i * ch, ch)], ib[b])
            pltpu.async_copy(vals.at[pl.ds(base + i * ch, ch)], vb[b], sems[b])

        def wait(b):
            pltpu.make_async_copy(
                vals.at[pl.ds(base, ch)], vb[b], sems[b]
            ).wait()

        for b in range(depth):
            start(b, b)

        @pl.loop(0, main // depth - 1)
        def _(j):
            k = j * depth
            for b in range(depth):
                wait(b)
                pltpu.sync_copy(vb[b], acc.at[ib[b]], add=True)
                start(k + b + depth, b)

        for b in range(depth):
            wait(b)
            pltpu.sync_copy(vb[b], acc.at[ib[b]], add=True)

        for i in range(main, chunks):
            start(i, 0)
            wait(0)
            pltpu.sync_copy(vb[0], acc.at[ib[0]], add=True)

        plsc.subcore_barrier()
        pltpu.sync_copy(
            acc.at[pl.ds(rbase, rpt)], out.at[cid, pl.ds(rbase, rpt)]
        )

    return scatter_k


def _make_pool():
    chunks = NP // (NW * CH)
    per_w = chunks * CH
    rpt = GP // NS

    @functools.partial(
        pl.kernel,
        out_type=(
            jax.ShapeDtypeStruct((NC, GP, H), jnp.float32),
            jax.ShapeDtypeStruct((NC, GP, H), jnp.float32),
        ),
        mesh=plsc.VectorSubcoreMesh(**_MESH),
        scratch_types=[
            pltpu.VMEM((CH,), jnp.int32),
            pltpu.VMEM((CH, H), jnp.float32),
            pltpu.VMEM((CH, H), jnp.float32),
            pltpu.VMEM_SHARED((GP, H), jnp.float32),
            pltpu.VMEM_SHARED((GP, H), jnp.float32),
            pltpu.SemaphoreType.DMA,
        ],
    )
    def pool_k(vals, idx, zeros_c, ones_c, out_s, out_n, idx_v, vals_v,
               ones_v, acc_s, acc_n, sem):
        cid = lax.axis_index("c")
        sid = lax.axis_index("s")
        rbase = sid * rpt
        pltpu.sync_copy(zeros_c.at[pl.ds(0, CH)], vals_v)
        pltpu.sync_copy(ones_c.at[pl.ds(0, CH)], ones_v)
        pltpu.sync_copy(vals_v.at[pl.ds(0, rpt)], acc_s.at[pl.ds(rbase, rpt)])
        pltpu.sync_copy(vals_v.at[pl.ds(0, rpt)], acc_n.at[pl.ds(rbase, rpt)])
        plsc.subcore_barrier()

        base = _wid() * per_w

        @pl.loop(0, chunks)
        def _(i):
            off = base + i * CH
            pltpu.sync_copy(idx.at[pl.ds(off, CH)], idx_v)
            pltpu.sync_copy(vals.at[pl.ds(off, CH)], vals_v)
            pltpu.sync_copy(vals_v, acc_s.at[idx_v], add=True)
            pltpu.sync_copy(ones_v, acc_n.at[idx_v], add=True)

        plsc.subcore_barrier()
        pltpu.sync_copy(acc_s.at[pl.ds(rbase, rpt)], out_s.at[cid, pl.ds(rbase, rpt)])
        pltpu.sync_copy(acc_n.at[pl.ds(rbase, rpt)], out_n.at[cid, pl.ds(rbase, rpt)])

    return pool_k


CE = 2560


def _edge_tc(hdhs2, ea, Wc, bc):
    nb = E // CE

    def body(hh_ref, ea_ref, w_ref, b_ref, m_ref):
        w = w_ref[...]
        hd = hh_ref[0].astype(jnp.bfloat16)
        hs = hh_ref[1].astype(jnp.bfloat16)
        eab = ea_ref[...].astype(jnp.bfloat16)
        acc = jnp.dot(hd, w[0:H], preferred_element_type=jnp.float32)
        acc += jnp.dot(hs, w[H:2 * H], preferred_element_type=jnp.float32)
        acc += jnp.dot(eab, w[2 * H:], preferred_element_type=jnp.float32)
        acc += b_ref[...]
        f = acc[:, :H]
        s = acc[:, H:]
        sig = 0.5 + 0.5 * jnp.tanh(0.5 * f)
        sp = jnp.maximum(s, 0.0) + jnp.log1p(jnp.exp(-jnp.abs(s)))
        m_ref[...] = sig * sp

    return pl.pallas_call(
        body,
        grid=(nb,),
        in_specs=[
            pl.BlockSpec((2, CE, H), lambda i: (0, i, 0)),
            pl.BlockSpec((CE, R), lambda i: (i, 0)),
            pl.BlockSpec((2 * H + R, 2 * H), lambda i: (0, 0)),
            pl.BlockSpec((1, 2 * H), lambda i: (0, 0)),
        ],
        out_specs=pl.BlockSpec((CE, H), lambda i: (i, 0)),
        out_shape=jax.ShapeDtypeStruct((E, H), jnp.float32),
    )(hdhs2, ea, Wc, bc)


CB = 1024


def _combine_tc(h, parts):
    nb = NP // CB

    def body(h_ref, p_ref, o_ref):
        o_ref[...] = jnp.maximum(h_ref[...] + p_ref[0] + p_ref[1], 0.0)

    return pl.pallas_call(
        body,
        grid=(nb,),
        in_specs=[
            pl.BlockSpec((CB, H), lambda i: (i, 0)),
            pl.BlockSpec((2, CB, H), lambda i: (0, i, 0)),
        ],
        out_specs=pl.BlockSpec((CB, H), lambda i: (i, 0)),
        out_shape=jax.ShapeDtypeStruct((NP, H), jnp.float32),
    )(h, parts)


def _final_tc(sums, cnts, Wl, bl):
    def body(s_ref, c_ref, w_ref, b_ref, o_ref):
        ssum = s_ref[0] + s_ref[1]
        cnt = c_ref[0] + c_ref[1]
        pooled = ssum / jnp.maximum(cnt, 1.0)
        o_ref[...] = (
            jnp.dot(pooled, w_ref[...], preferred_element_type=jnp.float32)
            + b_ref[...]
        )

    return pl.pallas_call(
        body,
        in_specs=[
            pl.BlockSpec((NC, GP, H), lambda: (0, 0, 0)),
            pl.BlockSpec((NC, GP, H), lambda: (0, 0, 0)),
            pl.BlockSpec((H, H), lambda: (0, 0)),
            pl.BlockSpec((1, H), lambda: (0, 0)),
        ],
        out_specs=pl.BlockSpec((GP, H), lambda: (0, 0)),
        out_shape=jax.ShapeDtypeStruct((GP, H), jnp.float32),
    )(sums, cnts, Wl, bl.reshape(1, H))


_gather_emb = _make_gather(NP, depth=2)
_gather_edges = _make_gather(2 * E, depth=5)
_scatter_edges = _make_scatter(E, NP, depth=2, ch=80)
_pool = _make_pool()


def kernel(x, edge_index, edge_attr, batch, emb, Wf1, bf1, Ws1, bs1, Wf2, bf2,
           Ws2, bs2, Wf3, bf3, Ws3, bs3, Wl, bl):
    x = x.astype(jnp.int32)
    src = edge_index[0].astype(jnp.int32)
    dst = edge_index[1].astype(jnp.int32)
    batch = batch.astype(jnp.int32)

    xpad = jnp.pad(x, (0, NP - N))
    bpad = jnp.pad(batch, (0, NP - N), constant_values=G)
    eidx = jnp.concatenate([dst, src])
    zeros_c = jnp.zeros((CH, H), jnp.float32)
    ones_c = jnp.ones((CH, H), jnp.float32)

    h = _gather_emb(emb, xpad)

    layers = ((Wf1, bf1, Ws1, bs1), (Wf2, bf2, Ws2, bs2), (Wf3, bf3, Ws3, bs3))
    for Wf, bf, Ws, bs in layers:
        Wc = jnp.concatenate([Wf, Ws], axis=1).astype(jnp.bfloat16)
        bc = jnp.concatenate([bf, bs]).reshape(1, 2 * H)
        hdhs = _gather_edges(h, eidx).reshape(2, E, H)
        m = _edge_tc(hdhs, edge_attr, Wc, bc)
        parts = _scatter_edges(m, dst, zeros_c)
        h = _combine_tc(h, parts)

    sums, cnts = _pool(h, bpad, zeros_c, ones_c)
    out = _final_tc(sums, cnts, Wl, bl)
    return out[:G]


# speedup vs baseline: 1.5034x; 1.0998x over previous
import functools

import jax
import jax.numpy as jnp
from jax import lax
from jax.experimental import pallas as pl
from jax.experimental.pallas import tpu as pltpu
from jax.experimental.pallas import tpu_sc as plsc

N = 10000
E = 320000
H = 128
R = 32
G = 64

NC = 2
NS = 16
NW = NC * NS

NP = 10240
CH = 80
GP = 128

_MESH = dict(core_axis_name="c", subcore_axis_name="s")


def _wid():
    return lax.axis_index("s") * NC + lax.axis_index("c")


def _make_gather(B, depth):
    assert B % (NW * CH) == 0
    chunks = B // (NW * CH)
    per_w = chunks * CH
    assert chunks % depth == 0 and chunks // depth >= 2

    @functools.partial(
        pl.kernel,
        out_type=jax.ShapeDtypeStruct((B, H), jnp.float32),
        mesh=plsc.VectorSubcoreMesh(**_MESH),
        scratch_types=[
            pltpu.VMEM((per_w,), jnp.int32),
            *([pltpu.VMEM((CH, H), jnp.float32)] * depth),
            *([pltpu.SemaphoreType.DMA] * depth),
        ],
    )
    def gather_k(table, idx, out, idx_v, *bufs):
        rb = bufs[:depth]
        sems = bufs[depth:]
        base = _wid() * per_w
        pltpu.sync_copy(idx.at[pl.ds(base, per_w)], idx_v)

        def start(i, b):
            pltpu.async_copy(
                table.at[idx_v.at[pl.ds(i * CH, CH)]], rb[b], sems[b]
            )

        def wait(b):
            pltpu.make_async_copy(
                table.at[idx_v.at[pl.ds(0, CH)]], rb[b], sems[b]
            ).wait()

        for b in range(depth):
            start(b, b)

        @pl.loop(0, chunks // depth - 1)
        def _(j):
            k = j * depth
            for b in range(depth):
                wait(b)
                pltpu.sync_copy(rb[b], out.at[pl.ds(base + (k + b) * CH, CH)])
                start(k + b + depth, b)

        k_last = chunks - depth
        for b in range(depth):
            wait(b)
            pltpu.sync_copy(rb[b], out.at[pl.ds(base + (k_last + b) * CH, CH)])

    return gather_k


def _make_scatter(B, nseg, depth, ch):
    assert B % (NW * ch) == 0 and nseg % (NS * 8) == 0
    chunks = B // (NW * ch)
    per_w = chunks * ch
    main = (chunks // depth) * depth
    assert main // depth >= 2
    rpt = nseg // NS

    @functools.partial(
        pl.kernel,
        out_type=jax.ShapeDtypeStruct((NC, nseg, H), jnp.float32),
        mesh=plsc.VectorSubcoreMesh(**_MESH),
        scratch_types=[
            *([pltpu.VMEM((ch,), jnp.int32)] * depth),
            *([pltpu.VMEM((ch, H), jnp.float32)] * depth),
            pltpu.VMEM_SHARED((nseg, H), jnp.float32),
            *([pltpu.SemaphoreType.DMA] * depth),
        ],
    )
    def scatter_k(vals, idx, zeros_c, out, *rest):
        ib = rest[:depth]
        vb = rest[depth:2 * depth]
        acc = rest[2 * depth]
        sems = rest[2 * depth + 1:]
        cid = lax.axis_index("c")
        sid = lax.axis_index("s")
        rbase = sid * rpt
        pltpu.sync_copy(zeros_c.at[pl.ds(0, ch)], vb[0])
        nz = (rpt + ch - 1) // ch
        for j in range(nz):
            rows = min(ch, rpt - j * ch)
            pltpu.sync_copy(
                vb[0].at[pl.ds(0, rows)], acc.at[pl.ds(rbase + j * ch, rows)]
            )
        plsc.subcore_barrier()

        base = _wid() * per_w

        def start(i, b):
            pltpu.sync_copy(idx.at[pl.ds(base + i * ch, ch)], ib[b])
            pltpu.async_copy(vals.at[pl.ds(base + i * ch, ch)], vb[b], sems[b])

        def wait(b):
            pltpu.make_async_copy(
                vals.at[pl.ds(base, ch)], vb[b], sems[b]
            ).wait()

        for b in range(depth):
            start(b, b)

        @pl.loop(0, main // depth - 1)
        def _(j):
            k = j * depth
            for b in range(depth):
                wait(b)
                pltpu.sync_copy(vb[b], acc.at[ib[b]], add=True)
                start(k + b + depth, b)

        for b in range(depth):
            wait(b)
            pltpu.sync_copy(vb[b], acc.at[ib[b]], add=True)

        for i in range(main, chunks):
            start(i, 0)
            wait(0)
            pltpu.sync_copy(vb[0], acc.at[ib[0]], add=True)

        plsc.subcore_barrier()
        pltpu.sync_copy(
            acc.at[pl.ds(rbase, rpt)], out.at[cid, pl.ds(rbase, rpt)]
        )

    return scatter_k


def _make_pool():
    chunks = NP // (NW * CH)
    per_w = chunks * CH
    rpt = GP // NS

    @functools.partial(
        pl.kernel,
        out_type=(
            jax.ShapeDtypeStruct((NC, GP, H), jnp.float32),
            jax.ShapeDtypeStruct((NC, GP, H), jnp.float32),
        ),
        mesh=plsc.VectorSubcoreMesh(**_MESH),
        scratch_types=[
            pltpu.VMEM((CH,), jnp.int32),
            pltpu.VMEM((CH, H), jnp.float32),
            pltpu.VMEM((CH, H), jnp.float32),
            pltpu.VMEM_SHARED((GP, H), jnp.float32),
            pltpu.VMEM_SHARED((GP, H), jnp.float32),
            pltpu.SemaphoreType.DMA,
        ],
    )
    def pool_k(vals, idx, zeros_c, ones_c, out_s, out_n, idx_v, vals_v,
               ones_v, acc_s, acc_n, sem):
        cid = lax.axis_index("c")
        sid = lax.axis_index("s")
        rbase = sid * rpt
        pltpu.sync_copy(zeros_c.at[pl.ds(0, CH)], vals_v)
        pltpu.sync_copy(ones_c.at[pl.ds(0, CH)], ones_v)
        pltpu.sync_copy(vals_v.at[pl.ds(0, rpt)], acc_s.at[pl.ds(rbase, rpt)])
        pltpu.sync_copy(vals_v.at[pl.ds(0, rpt)], acc_n.at[pl.ds(rbase, rpt)])
        plsc.subcore_barrier()

        base = _wid() * per_w

        @pl.loop(0, chunks)
        def _(i):
            off = base + i * CH
            pltpu.sync_copy(idx.at[pl.ds(off, CH)], idx_v)
            pltpu.sync_copy(vals.at[pl.ds(off, CH)], vals_v)
            pltpu.sync_copy(vals_v, acc_s.at[idx_v], add=True)
            pltpu.sync_copy(ones_v, acc_n.at[idx_v], add=True)

        plsc.subcore_barrier()
        pltpu.sync_copy(acc_s.at[pl.ds(rbase, rpt)], out_s.at[cid, pl.ds(rbase, rpt)])
        pltpu.sync_copy(acc_n.at[pl.ds(rbase, rpt)], out_n.at[cid, pl.ds(rbase, rpt)])

    return pool_k


CE = 2560


def _edge_tc(hdhs2, ea, Wc, bc):
    ne = hdhs2.shape[1]
    nb = ne // CE

    def body(hh_ref, ea_ref, w_ref, b_ref, m_ref):
        w = w_ref[...]
        hd = hh_ref[0].astype(jnp.bfloat16)
        hs = hh_ref[1].astype(jnp.bfloat16)
        eab = ea_ref[...].astype(jnp.bfloat16)
        acc = jnp.dot(hd, w[0:H], preferred_element_type=jnp.float32)
        acc += jnp.dot(hs, w[H:2 * H], preferred_element_type=jnp.float32)
        acc += jnp.dot(eab, w[2 * H:], preferred_element_type=jnp.float32)
        acc += b_ref[...]
        f = acc[:, :H]
        s = acc[:, H:]
        sig = 0.5 + 0.5 * jnp.tanh(0.5 * f)
        sp = jnp.maximum(s, 0.0) + jnp.log1p(jnp.exp(-jnp.abs(s)))
        m_ref[...] = sig * sp

    return pl.pallas_call(
        body,
        grid=(nb,),
        in_specs=[
            pl.BlockSpec((2, CE, H), lambda i: (0, i, 0)),
            pl.BlockSpec((CE, R), lambda i: (i, 0)),
            pl.BlockSpec((2 * H + R, 2 * H), lambda i: (0, 0)),
            pl.BlockSpec((1, 2 * H), lambda i: (0, 0)),
        ],
        out_specs=pl.BlockSpec((CE, H), lambda i: (i, 0)),
        out_shape=jax.ShapeDtypeStruct((ne, H), jnp.float32),
    )(hdhs2, ea, Wc, bc)


CB = 1024


def _combine_tc(h, parts_a, parts_b):
    nb = NP // CB

    def body(h_ref, pa_ref, pb_ref, o_ref):
        o_ref[...] = jnp.maximum(
            h_ref[...] + (pa_ref[0] + pa_ref[1]) + (pb_ref[0] + pb_ref[1]), 0.0
        )

    return pl.pallas_call(
        body,
        grid=(nb,),
        in_specs=[
            pl.BlockSpec((CB, H), lambda i: (i, 0)),
            pl.BlockSpec((2, CB, H), lambda i: (0, i, 0)),
            pl.BlockSpec((2, CB, H), lambda i: (0, i, 0)),
        ],
        out_specs=pl.BlockSpec((CB, H), lambda i: (i, 0)),
        out_shape=jax.ShapeDtypeStruct((NP, H), jnp.float32),
    )(h, parts_a, parts_b)


def _final_tc(sums, cnts, Wl, bl):
    def body(s_ref, c_ref, w_ref, b_ref, o_ref):
        ssum = s_ref[0] + s_ref[1]
        cnt = c_ref[0] + c_ref[1]
        pooled = ssum / jnp.maximum(cnt, 1.0)
        o_ref[...] = (
            jnp.dot(pooled, w_ref[...], preferred_element_type=jnp.float32)
            + b_ref[...]
        )

    return pl.pallas_call(
        body,
        in_specs=[
            pl.BlockSpec((NC, GP, H), lambda: (0, 0, 0)),
            pl.BlockSpec((NC, GP, H), lambda: (0, 0, 0)),
            pl.BlockSpec((H, H), lambda: (0, 0)),
            pl.BlockSpec((1, H), lambda: (0, 0)),
        ],
        out_specs=pl.BlockSpec((GP, H), lambda: (0, 0)),
        out_shape=jax.ShapeDtypeStruct((GP, H), jnp.float32),
    )(sums, cnts, Wl, bl.reshape(1, H))


EA = 192000
EB = E - EA

_gather_emb = _make_gather(NP, depth=2)
_gather_a = _make_gather(2 * EA, depth=5)
_gather_b = _make_gather(2 * EB, depth=5)
_scatter_a = _make_scatter(EA, NP, depth=2, ch=80)
_scatter_b = _make_scatter(EB, NP, depth=2, ch=80)
_pool = _make_pool()


def kernel(x, edge_index, edge_attr, batch, emb, Wf1, bf1, Ws1, bs1, Wf2, bf2,
           Ws2, bs2, Wf3, bf3, Ws3, bs3, Wl, bl):
    x = x.astype(jnp.int32)
    src = edge_index[0].astype(jnp.int32)
    dst = edge_index[1].astype(jnp.int32)
    batch = batch.astype(jnp.int32)

    xpad = jnp.pad(x, (0, NP - N))
    bpad = jnp.pad(batch, (0, NP - N), constant_values=G)
    dst_a, dst_b = dst[:EA], dst[EA:]
    eidx_a = jnp.concatenate([dst_a, src[:EA]])
    eidx_b = jnp.concatenate([dst_b, src[EA:]])
    ea_a, ea_b = edge_attr[:EA], edge_attr[EA:]
    zeros_c = jnp.zeros((CH, H), jnp.float32)
    ones_c = jnp.ones((CH, H), jnp.float32)

    h = _gather_emb(emb, xpad)

    layers = ((Wf1, bf1, Ws1, bs1), (Wf2, bf2, Ws2, bs2), (Wf3, bf3, Ws3, bs3))
    for Wf, bf, Ws, bs in layers:
        Wc = jnp.concatenate([Wf, Ws], axis=1).astype(jnp.bfloat16)
        bc = jnp.concatenate([bf, bs]).reshape(1, 2 * H)
        ga = _gather_a(h, eidx_a).reshape(2, EA, H)
        ma = _edge_tc(ga, ea_a, Wc, bc)
        gb = _gather_b(h, eidx_b).reshape(2, EB, H)
        pa = _scatter_a(ma, dst_a, zeros_c)
        mb = _edge_tc(gb, ea_b, Wc, bc)
        pb = _scatter_b(mb, dst_b, zeros_c)
        h = _combine_tc(h, pa, pb)

    sums, cnts = _pool(h, bpad, zeros_c, ones_c)
    out = _final_tc(sums, cnts, Wl, bl)
    return out[:G]


# speedup vs baseline: 1.5246x; 1.0141x over previous
import functools

import jax
import jax.numpy as jnp
from jax import lax
from jax.experimental import pallas as pl
from jax.experimental.pallas import tpu as pltpu
from jax.experimental.pallas import tpu_sc as plsc

N = 10000
E = 320000
H = 128
R = 32
G = 64

NC = 2
NS = 16
NW = NC * NS

NP = 10240
CH = 80
GP = 128

_MESH = dict(core_axis_name="c", subcore_axis_name="s")


def _wid():
    return lax.axis_index("s") * NC + lax.axis_index("c")


def _make_gather(B, depth):
    assert B % (NW * CH) == 0
    chunks = B // (NW * CH)
    per_w = chunks * CH
    assert chunks % depth == 0 and chunks // depth >= 2

    @functools.partial(
        pl.kernel,
        out_type=jax.ShapeDtypeStruct((B, H), jnp.float32),
        mesh=plsc.VectorSubcoreMesh(**_MESH),
        scratch_types=[
            pltpu.VMEM((per_w,), jnp.int32),
            *([pltpu.VMEM((CH, H), jnp.float32)] * depth),
            *([pltpu.SemaphoreType.DMA] * depth),
        ],
    )
    def gather_k(table, idx, out, idx_v, *bufs):
        rb = bufs[:depth]
        sems = bufs[depth:]
        base = _wid() * per_w
        pltpu.sync_copy(idx.at[pl.ds(base, per_w)], idx_v)

        def start(i, b):
            pltpu.async_copy(
                table.at[idx_v.at[pl.ds(i * CH, CH)]], rb[b], sems[b]
            )

        def wait(b):
            pltpu.make_async_copy(
                table.at[idx_v.at[pl.ds(0, CH)]], rb[b], sems[b]
            ).wait()

        for b in range(depth):
            start(b, b)

        @pl.loop(0, chunks // depth - 1)
        def _(j):
            k = j * depth
            for b in range(depth):
                wait(b)
                pltpu.sync_copy(rb[b], out.at[pl.ds(base + (k + b) * CH, CH)])
                start(k + b + depth, b)

        k_last = chunks - depth
        for b in range(depth):
            wait(b)
            pltpu.sync_copy(rb[b], out.at[pl.ds(base + (k_last + b) * CH, CH)])

    return gather_k


def _make_scatter(B, nseg, depth, ch):
    assert B % (NW * ch) == 0 and nseg % (NS * 8) == 0
    chunks = B // (NW * ch)
    per_w = chunks * ch
    main = (chunks // depth) * depth
    assert main // depth >= 2
    rpt = nseg // NS

    @functools.partial(
        pl.kernel,
        out_type=jax.ShapeDtypeStruct((NC, nseg, H), jnp.float32),
        mesh=plsc.VectorSubcoreMesh(**_MESH),
        scratch_types=[
            *([pltpu.VMEM((ch,), jnp.int32)] * depth),
            *([pltpu.VMEM((ch, H), jnp.float32)] * depth),
            pltpu.VMEM_SHARED((nseg, H), jnp.float32),
            *([pltpu.SemaphoreType.DMA] * depth),
        ],
    )
    def scatter_k(vals, idx, zeros_c, out, *rest):
        ib = rest[:depth]
        vb = rest[depth:2 * depth]
        acc = rest[2 * depth]
        sems = rest[2 * depth + 1:]
        cid = lax.axis_index("c")
        sid = lax.axis_index("s")
        rbase = sid * rpt
        pltpu.sync_copy(zeros_c.at[pl.ds(0, ch)], vb[0])
        nz = (rpt + ch - 1) // ch
        for j in range(nz):
            rows = min(ch, rpt - j * ch)
            pltpu.sync_copy(
                vb[0].at[pl.ds(0, rows)], acc.at[pl.ds(rbase + j * ch, rows)]
            )
        plsc.subcore_barrier()

        base = _wid() * per_w

        def start(i, b):
            pltpu.sync_copy(idx.at[pl.ds(base + i * ch, ch)], ib[b])
            pltpu.async_copy(vals.at[pl.ds(base + i * ch, ch)], vb[b], sems[b])

        def wait(b):
            pltpu.make_async_copy(
                vals.at[pl.ds(base, ch)], vb[b], sems[b]
            ).wait()

        for b in range(depth):
            start(b, b)

        @pl.loop(0, main // depth - 1)
        def _(j):
            k = j * depth
            for b in range(depth):
                wait(b)
                pltpu.sync_copy(vb[b], acc.at[ib[b]], add=True)
                start(k + b + depth, b)

        for b in range(depth):
            wait(b)
            pltpu.sync_copy(vb[b], acc.at[ib[b]], add=True)

        for i in range(main, chunks):
            start(i, 0)
            wait(0)
            pltpu.sync_copy(vb[0], acc.at[ib[0]], add=True)

        plsc.subcore_barrier()
        pltpu.sync_copy(
            acc.at[pl.ds(rbase, rpt)], out.at[cid, pl.ds(rbase, rpt)]
        )

    return scatter_k


def _make_pool():
    chunks = NP // (NW * CH)
    per_w = chunks * CH
    rpt = GP // NS

    @functools.partial(
        pl.kernel,
        out_type=(
            jax.ShapeDtypeStruct((NC, GP, H), jnp.float32),
            jax.ShapeDtypeStruct((NC, GP, H), jnp.float32),
        ),
        mesh=plsc.VectorSubcoreMesh(**_MESH),
        scratch_types=[
            pltpu.VMEM((CH,), jnp.int32),
            pltpu.VMEM((CH, H), jnp.float32),
            pltpu.VMEM((CH, H), jnp.float32),
            pltpu.VMEM_SHARED((GP, H), jnp.float32),
            pltpu.VMEM_SHARED((GP, H), jnp.float32),
            pltpu.SemaphoreType.DMA,
        ],
    )
    def pool_k(vals, idx, zeros_c, ones_c, out_s, out_n, idx_v, vals_v,
               ones_v, acc_s, acc_n, sem):
        cid = lax.axis_index("c")
        sid = lax.axis_index("s")
        rbase = sid * rpt
        pltpu.sync_copy(zeros_c.at[pl.ds(0, CH)], vals_v)
        pltpu.sync_copy(ones_c.at[pl.ds(0, CH)], ones_v)
        pltpu.sync_copy(vals_v.at[pl.ds(0, rpt)], acc_s.at[pl.ds(rbase, rpt)])
        pltpu.sync_copy(vals_v.at[pl.ds(0, rpt)], acc_n.at[pl.ds(rbase, rpt)])
        plsc.subcore_barrier()

        base = _wid() * per_w

        @pl.loop(0, chunks)
        def _(i):
            off = base + i * CH
            pltpu.sync_copy(idx.at[pl.ds(off, CH)], idx_v)
            pltpu.sync_copy(vals.at[pl.ds(off, CH)], vals_v)
            pltpu.sync_copy(vals_v, acc_s.at[idx_v], add=True)
            pltpu.sync_copy(ones_v, acc_n.at[idx_v], add=True)

        plsc.subcore_barrier()
        pltpu.sync_copy(acc_s.at[pl.ds(rbase, rpt)], out_s.at[cid, pl.ds(rbase, rpt)])
        pltpu.sync_copy(acc_n.at[pl.ds(rbase, rpt)], out_n.at[cid, pl.ds(rbase, rpt)])

    return pool_k


CE = 2560


def _edge_tc(hdhs2, ea, Wc, bc):
    ne = hdhs2.shape[1]
    nb = ne // CE

    def body(hh_ref, ea_ref, w_ref, b_ref, m_ref):
        w = w_ref[...]
        hd = hh_ref[0].astype(jnp.bfloat16)
        hs = hh_ref[1].astype(jnp.bfloat16)
        eab = ea_ref[...].astype(jnp.bfloat16)
        acc = jnp.dot(hd, w[0:H], preferred_element_type=jnp.float32)
        acc += jnp.dot(hs, w[H:2 * H], preferred_element_type=jnp.float32)
        acc += jnp.dot(eab, w[2 * H:], preferred_element_type=jnp.float32)
        acc += b_ref[...]
        f = acc[:, :H]
        s = acc[:, H:]
        sig = 0.5 + 0.5 * jnp.tanh(0.5 * f)
        sp = jnp.maximum(s, 0.0) + jnp.log1p(jnp.exp(-jnp.abs(s)))
        m_ref[...] = sig * sp

    return pl.pallas_call(
        body,
        grid=(nb,),
        in_specs=[
            pl.BlockSpec((2, CE, H), lambda i: (0, i, 0)),
            pl.BlockSpec((CE, R), lambda i: (i, 0)),
            pl.BlockSpec((2 * H + R, 2 * H), lambda i: (0, 0)),
            pl.BlockSpec((1, 2 * H), lambda i: (0, 0)),
        ],
        out_specs=pl.BlockSpec((CE, H), lambda i: (i, 0)),
        out_shape=jax.ShapeDtypeStruct((ne, H), jnp.float32),
    )(hdhs2, ea, Wc, bc)


CB = 1024


def _combine_tc(h, parts_list):
    nb = NP // CB
    ns = len(parts_list)

    def body(h_ref, *refs):
        p_refs = refs[:ns]
        o_ref = refs[ns]
        acc = h_ref[...]
        for pr in p_refs:
            acc = acc + pr[0] + pr[1]
        o_ref[...] = jnp.maximum(acc, 0.0)

    return pl.pallas_call(
        body,
        grid=(nb,),
        in_specs=[pl.BlockSpec((CB, H), lambda i: (i, 0))]
        + [pl.BlockSpec((2, CB, H), lambda i: (0, i, 0))] * ns,
        out_specs=pl.BlockSpec((CB, H), lambda i: (i, 0)),
        out_shape=jax.ShapeDtypeStruct((NP, H), jnp.float32),
    )(h, *parts_list)


def _final_tc(sums, cnts, Wl, bl):
    def body(s_ref, c_ref, w_ref, b_ref, o_ref):
        ssum = s_ref[0] + s_ref[1]
        cnt = c_ref[0] + c_ref[1]
        pooled = ssum / jnp.maximum(cnt, 1.0)
        o_ref[...] = (
            jnp.dot(pooled, w_ref[...], preferred_element_type=jnp.float32)
            + b_ref[...]
        )

    return pl.pallas_call(
        body,
        in_specs=[
            pl.BlockSpec((NC, GP, H), lambda: (0, 0, 0)),
            pl.BlockSpec((NC, GP, H), lambda: (0, 0, 0)),
            pl.BlockSpec((H, H), lambda: (0, 0)),
            pl.BlockSpec((1, H), lambda: (0, 0)),
        ],
        out_specs=pl.BlockSpec((GP, H), lambda: (0, 0)),
        out_shape=jax.ShapeDtypeStruct((GP, H), jnp.float32),
    )(sums, cnts, Wl, bl.reshape(1, H))


ESPLITS = (102400, 102400, 115200)

_gather_emb = _make_gather(NP, depth=2)
_gather_s = tuple(_make_gather(2 * e, depth=5) for e in ESPLITS)
_scatter_s = tuple(_make_scatter(e, NP, depth=2, ch=80) for e in ESPLITS)
_pool = _make_pool()


def kernel(x, edge_index, edge_attr, batch, emb, Wf1, bf1, Ws1, bs1, Wf2, bf2,
           Ws2, bs2, Wf3, bf3, Ws3, bs3, Wl, bl):
    x = x.astype(jnp.int32)
    src = edge_index[0].astype(jnp.int32)
    dst = edge_index[1].astype(jnp.int32)
    batch = batch.astype(jnp.int32)

    xpad = jnp.pad(x, (0, NP - N))
    bpad = jnp.pad(batch, (0, NP - N), constant_values=G)
    offs = [0]
    for e in ESPLITS:
        offs.append(offs[-1] + e)
    dst_s = [dst[o:o + e] for o, e in zip(offs, ESPLITS)]
    eidx_s = [jnp.concatenate([dst[o:o + e], src[o:o + e]])
              for o, e in zip(offs, ESPLITS)]
    ea_s = [edge_attr[o:o + e] for o, e in zip(offs, ESPLITS)]
    zeros_c = jnp.zeros((CH, H), jnp.float32)
    ones_c = jnp.ones((CH, H), jnp.float32)

    h = _gather_emb(emb, xpad)

    layers = ((Wf1, bf1, Ws1, bs1), (Wf2, bf2, Ws2, bs2), (Wf3, bf3, Ws3, bs3))
    for Wf, bf, Ws, bs in layers:
        Wc = jnp.concatenate([Wf, Ws], axis=1).astype(jnp.bfloat16)
        bc = jnp.concatenate([bf, bs]).reshape(1, 2 * H)
        gs = []
        ms = []
        ps = []
        for i, e in enumerate(ESPLITS):
            gs.append(_gather_s[i](h, eidx_s[i]).reshape(2, e, H))
            ms.append(_edge_tc(gs[i], ea_s[i], Wc, bc))
            ps.append(_scatter_s[i](ms[i], dst_s[i], zeros_c))
        h = _combine_tc(h, ps)

    sums, cnts = _pool(h, bpad, zeros_c, ones_c)
    out = _final_tc(sums, cnts, Wl, bl)
    return out[:G]


# speedup vs baseline: 1.5312x; 1.0043x over previous
import functools

import jax
import jax.numpy as jnp
from jax import lax
from jax.experimental import pallas as pl
from jax.experimental.pallas import tpu as pltpu
from jax.experimental.pallas import tpu_sc as plsc

N = 10000
E = 320000
H = 128
R = 32
G = 64

NC = 2
NS = 16
NW = NC * NS

NP = 10240
CH = 80
GP = 128

_MESH = dict(core_axis_name="c", subcore_axis_name="s")


def _wid():
    return lax.axis_index("s") * NC + lax.axis_index("c")


def _make_gather(B, depth, ch=CH):
    assert B % (NW * ch) == 0
    chunks = B // (NW * ch)
    per_w = chunks * ch
    assert chunks % depth == 0 and chunks // depth >= 2

    @functools.partial(
        pl.kernel,
        out_type=jax.ShapeDtypeStruct((B, H), jnp.float32),
        mesh=plsc.VectorSubcoreMesh(**_MESH),
        scratch_types=[
            pltpu.VMEM((per_w,), jnp.int32),
            *([pltpu.VMEM((ch, H), jnp.float32)] * depth),
            *([pltpu.SemaphoreType.DMA] * depth),
        ],
    )
    def gather_k(table, idx, out, idx_v, *bufs):
        rb = bufs[:depth]
        sems = bufs[depth:]
        base = _wid() * per_w
        pltpu.sync_copy(idx.at[pl.ds(base, per_w)], idx_v)

        def start(i, b):
            pltpu.async_copy(
                table.at[idx_v.at[pl.ds(i * ch, ch)]], rb[b], sems[b]
            )

        def wait(b):
            pltpu.make_async_copy(
                table.at[idx_v.at[pl.ds(0, ch)]], rb[b], sems[b]
            ).wait()

        for b in range(depth):
            start(b, b)

        @pl.loop(0, chunks // depth - 1)
        def _(j):
            k = j * depth
            for b in range(depth):
                wait(b)
                pltpu.sync_copy(rb[b], out.at[pl.ds(base + (k + b) * ch, ch)])
                start(k + b + depth, b)

        k_last = chunks - depth
        for b in range(depth):
            wait(b)
            pltpu.sync_copy(rb[b], out.at[pl.ds(base + (k_last + b) * ch, ch)])

    return gather_k


def _make_scatter(B, nseg, depth, ch):
    assert B % (NW * ch) == 0 and nseg % (NS * 8) == 0
    chunks = B // (NW * ch)
    per_w = chunks * ch
    main = (chunks // depth) * depth
    assert main // depth >= 2
    rpt = nseg // NS

    @functools.partial(
        pl.kernel,
        out_type=jax.ShapeDtypeStruct((NC, nseg, H), jnp.float32),
        mesh=plsc.VectorSubcoreMesh(**_MESH),
        scratch_types=[
            *([pltpu.VMEM((ch,), jnp.int32)] * depth),
            *([pltpu.VMEM((ch, H), jnp.float32)] * depth),
            pltpu.VMEM_SHARED((nseg, H), jnp.float32),
            *([pltpu.SemaphoreType.DMA] * depth),
        ],
    )
    def scatter_k(vals, idx, zeros_c, out, *rest):
        ib = rest[:depth]
        vb = rest[depth:2 * depth]
        acc = rest[2 * depth]
        sems = rest[2 * depth + 1:]
        cid = lax.axis_index("c")
        sid = lax.axis_index("s")
        rbase = sid * rpt
        pltpu.sync_copy(zeros_c.at[pl.ds(0, ch)], vb[0])
        nz = (rpt + ch - 1) // ch
        for j in range(nz):
            rows = min(ch, rpt - j * ch)
            pltpu.sync_copy(
                vb[0].at[pl.ds(0, rows)], acc.at[pl.ds(rbase + j * ch, rows)]
            )
        plsc.subcore_barrier()

        base = _wid() * per_w

        def start(i, b):
            pltpu.sync_copy(idx.at[pl.ds(base + i * ch, ch)], ib[b])
            pltpu.async_copy(vals.at[pl.ds(base + i * ch, ch)], vb[b], sems[b])

        def wait(b):
            pltpu.make_async_copy(
                vals.at[pl.ds(base, ch)], vb[b], sems[b]
            ).wait()

        for b in range(depth):
            start(b, b)

        @pl.loop(0, main // depth - 1)
        def _(j):
            k = j * depth
            for b in range(depth):
                wait(b)
                pltpu.sync_copy(vb[b], acc.at[ib[b]], add=True)
                start(k + b + depth, b)

        for b in range(depth):
            wait(b)
            pltpu.sync_copy(vb[b], acc.at[ib[b]], add=True)

        for i in range(main, chunks):
            start(i, 0)
            wait(0)
            pltpu.sync_copy(vb[0], acc.at[ib[0]], add=True)

        plsc.subcore_barrier()
        pltpu.sync_copy(
            acc.at[pl.ds(rbase, rpt)], out.at[cid, pl.ds(rbase, rpt)]
        )

    return scatter_k


def _make_pool():
    chunks = NP // (NW * CH)
    per_w = chunks * CH
    rpt = GP // NS

    @functools.partial(
        pl.kernel,
        out_type=(
            jax.ShapeDtypeStruct((NC, GP, H), jnp.float32),
            jax.ShapeDtypeStruct((NC, GP, H), jnp.float32),
        ),
        mesh=plsc.VectorSubcoreMesh(**_MESH),
        scratch_types=[
            pltpu.VMEM((CH,), jnp.int32),
            pltpu.VMEM((CH, H), jnp.float32),
            pltpu.VMEM((CH, H), jnp.float32),
            pltpu.VMEM_SHARED((GP, H), jnp.float32),
            pltpu.VMEM_SHARED((GP, H), jnp.float32),
            pltpu.SemaphoreType.DMA,
        ],
    )
    def pool_k(vals, idx, zeros_c, ones_c, out_s, out_n, idx_v, vals_v,
               ones_v, acc_s, acc_n, sem):
        cid = lax.axis_index("c")
        sid = lax.axis_index("s")
        rbase = sid * rpt
        pltpu.sync_copy(zeros_c.at[pl.ds(0, CH)], vals_v)
        pltpu.sync_copy(ones_c.at[pl.ds(0, CH)], ones_v)
        pltpu.sync_copy(vals_v.at[pl.ds(0, rpt)], acc_s.at[pl.ds(rbase, rpt)])
        pltpu.sync_copy(vals_v.at[pl.ds(0, rpt)], acc_n.at[pl.ds(rbase, rpt)])
        plsc.subcore_barrier()

        base = _wid() * per_w

        @pl.loop(0, chunks)
        def _(i):
            off = base + i * CH
            pltpu.sync_copy(idx.at[pl.ds(off, CH)], idx_v)
            pltpu.sync_copy(vals.at[pl.ds(off, CH)], vals_v)
            pltpu.sync_copy(vals_v, acc_s.at[idx_v], add=True)
            pltpu.sync_copy(ones_v, acc_n.at[idx_v], add=True)

        plsc.subcore_barrier()
        pltpu.sync_copy(acc_s.at[pl.ds(rbase, rpt)], out_s.at[cid, pl.ds(rbase, rpt)])
        pltpu.sync_copy(acc_n.at[pl.ds(rbase, rpt)], out_n.at[cid, pl.ds(rbase, rpt)])

    return pool_k


CE = 2560


def _edge_tc(hdhs2, ea, Wc, bc):
    ne = hdhs2.shape[1]
    nb = ne // CE

    def body(hh_ref, ea_ref, w_ref, b_ref, m_ref):
        w = w_ref[...]
        hd = hh_ref[0].astype(jnp.bfloat16)
        hs = hh_ref[1].astype(jnp.bfloat16)
        eab = ea_ref[...].astype(jnp.bfloat16)
        acc = jnp.dot(hd, w[0:H], preferred_element_type=jnp.float32)
        acc += jnp.dot(hs, w[H:2 * H], preferred_element_type=jnp.float32)
        acc += jnp.dot(eab, w[2 * H:], preferred_element_type=jnp.float32)
        acc += b_ref[...]
        f = acc[:, :H]
        s = acc[:, H:]
        sig = 0.5 + 0.5 * jnp.tanh(0.5 * f)
        sp = jnp.maximum(s, 0.0) + jnp.log1p(jnp.exp(-jnp.abs(s)))
        m_ref[...] = sig * sp

    return pl.pallas_call(
        body,
        grid=(nb,),
        in_specs=[
            pl.BlockSpec((2, CE, H), lambda i: (0, i, 0)),
            pl.BlockSpec((CE, R), lambda i: (i, 0)),
            pl.BlockSpec((2 * H + R, 2 * H), lambda i: (0, 0)),
            pl.BlockSpec((1, 2 * H), lambda i: (0, 0)),
        ],
        out_specs=pl.BlockSpec((CE, H), lambda i: (i, 0)),
        out_shape=jax.ShapeDtypeStruct((ne, H), jnp.float32),
    )(hdhs2, ea, Wc, bc)


CB = 1024


def _combine_tc(h, parts_list):
    nb = NP // CB
    ns = len(parts_list)

    def body(h_ref, *refs):
        p_refs = refs[:ns]
        o_ref = refs[ns]
        acc = h_ref[...]
        for pr in p_refs:
            acc = acc + pr[0] + pr[1]
        o_ref[...] = jnp.maximum(acc, 0.0)

    return pl.pallas_call(
        body,
        grid=(nb,),
        in_specs=[pl.BlockSpec((CB, H), lambda i: (i, 0))]
        + [pl.BlockSpec((2, CB, H), lambda i: (0, i, 0))] * ns,
        out_specs=pl.BlockSpec((CB, H), lambda i: (i, 0)),
        out_shape=jax.ShapeDtypeStruct((NP, H), jnp.float32),
    )(h, *parts_list)


def _final_tc(sums, cnts, Wl, bl):
    def body(s_ref, c_ref, w_ref, b_ref, o_ref):
        ssum = s_ref[0] + s_ref[1]
        cnt = c_ref[0] + c_ref[1]
        pooled = ssum / jnp.maximum(cnt, 1.0)
        o_ref[...] = (
            jnp.dot(pooled, w_ref[...], preferred_element_type=jnp.float32)
            + b_ref[...]
        )

    return pl.pallas_call(
        body,
        in_specs=[
            pl.BlockSpec((NC, GP, H), lambda: (0, 0, 0)),
            pl.BlockSpec((NC, GP, H), lambda: (0, 0, 0)),
            pl.BlockSpec((H, H), lambda: (0, 0)),
            pl.BlockSpec((1, H), lambda: (0, 0)),
        ],
        out_specs=pl.BlockSpec((GP, H), lambda: (0, 0)),
        out_shape=jax.ShapeDtypeStruct((GP, H), jnp.float32),
    )(sums, cnts, Wl, bl.reshape(1, H))


ESPLITS = (102400, 102400, 115200)

_gather_emb = _make_gather(NP, depth=2)
_gather_s = tuple(_make_gather(2 * e, depth=5, ch=160) for e in ESPLITS)
_scatter_s = tuple(_make_scatter(e, NP, depth=2, ch=(160 if e % (NW * 160) == 0 else 80)) for e in ESPLITS)
_pool = _make_pool()


def kernel(x, edge_index, edge_attr, batch, emb, Wf1, bf1, Ws1, bs1, Wf2, bf2,
           Ws2, bs2, Wf3, bf3, Ws3, bs3, Wl, bl):
    x = x.astype(jnp.int32)
    src = edge_index[0].astype(jnp.int32)
    dst = edge_index[1].astype(jnp.int32)
    batch = batch.astype(jnp.int32)

    xpad = jnp.pad(x, (0, NP - N))
    bpad = jnp.pad(batch, (0, NP - N), constant_values=G)
    offs = [0]
    for e in ESPLITS:
        offs.append(offs[-1] + e)
    dst_s = [dst[o:o + e] for o, e in zip(offs, ESPLITS)]
    eidx_s = [jnp.concatenate([dst[o:o + e], src[o:o + e]])
              for o, e in zip(offs, ESPLITS)]
    ea_s = [edge_attr[o:o + e] for o, e in zip(offs, ESPLITS)]
    zeros_c = jnp.zeros((CH, H), jnp.float32)
    ones_c = jnp.ones((CH, H), jnp.float32)

    h = _gather_emb(emb, xpad)

    layers = ((Wf1, bf1, Ws1, bs1), (Wf2, bf2, Ws2, bs2), (Wf3, bf3, Ws3, bs3))
    for Wf, bf, Ws, bs in layers:
        Wc = jnp.concatenate([Wf, Ws], axis=1).astype(jnp.bfloat16)
        bc = jnp.concatenate([bf, bs]).reshape(1, 2 * H)
        gs = []
        ms = []
        ps = []
        for i, e in enumerate(ESPLITS):
            gs.append(_gather_s[i](h, eidx_s[i]).reshape(2, e, H))
            ms.append(_edge_tc(gs[i], ea_s[i], Wc, bc))
            ps.append(_scatter_s[i](ms[i], dst_s[i], zeros_c))
        h = _combine_tc(h, ps)

    sums, cnts = _pool(h, bpad, zeros_c, ones_c)
    out = _final_tc(sums, cnts, Wl, bl)
    return out[:G]
